# baseline (speedup 1.0000x reference)
import functools
import math

import jax
import jax.numpy as jnp
from jax import lax
from jax.experimental import pallas as pl
from jax.experimental.pallas import tpu as pltpu
from jax.experimental.pallas import tpu_sc as plsc

B = 16384
STATE_DIM = 256
ENC_DIM = 128
N_CAND = 1000
N_PAD = 1024
SLATE = 50
BR = 256

_IMIN = -2147483648
L = 16
NW = 32
RPW = B // NW
RT = 8
NT = RPW // RT


def _tc_body(x_ref, w_ref, b_ref, enc_ref,
             scores_ref, probs_ref, ha_ref, keys_ref, md_ref):
    x = x_ref[...]
    ha = jnp.dot(x, w_ref[...], preferred_element_type=jnp.float32)
    ha = ha + b_ref[...][None, :]
    ha_ref[...] = ha
    s = lax.dot_general(
        ha, enc_ref[...], (((1,), (1,)), ((), ())),
        preferred_element_type=jnp.float32,
    ) * (1.0 / math.sqrt(ENC_DIM))
    col = lax.broadcasted_iota(jnp.int32, (BR, N_PAD), 1)
    valid = col < N_CAND
    s_m = jnp.where(valid, s, -jnp.inf)
    m = jnp.max(s_m, axis=1, keepdims=True)
    e = jnp.exp(s_m - m)
    denom = jnp.sum(e, axis=1, keepdims=True)
    probs_ref[...] = (e / denom)[:, :N_CAND]
    scores_ref[...] = s[:, :N_CAND]
    bits = lax.bitcast_convert_type(s, jnp.int32)
    key = jnp.where(bits >= 0, bits, bits ^ jnp.int32(0x7FFFFFFF))
    keys_ref[...] = jnp.where(valid, key, jnp.int32(_IMIN))
    md_ref[...] = jnp.concatenate(
        [m, denom, jnp.zeros((BR, 14), jnp.float32)], axis=1)


def _tc_call(user_state, W_h, b_h, enc_pad):
    return pl.pallas_call(
        _tc_body,
        grid=(B // BR,),
        in_specs=[
            pl.BlockSpec((BR, STATE_DIM), lambda i: (i, 0)),
            pl.BlockSpec((STATE_DIM, ENC_DIM), lambda i: (0, 0)),
            pl.BlockSpec((ENC_DIM,), lambda i: (0,)),
            pl.BlockSpec((N_PAD, ENC_DIM), lambda i: (0, 0)),
        ],
        out_specs=[
            pl.BlockSpec((BR, N_CAND), lambda i: (i, 0)),
            pl.BlockSpec((BR, N_CAND), lambda i: (i, 0)),
            pl.BlockSpec((BR, ENC_DIM), lambda i: (i, 0)),
            pl.BlockSpec((BR, N_PAD), lambda i: (i, 0)),
            pl.BlockSpec((BR, 16), lambda i: (i, 0)),
        ],
        out_shape=[
            jax.ShapeDtypeStruct((B, N_CAND), jnp.float32),
            jax.ShapeDtypeStruct((B, N_CAND), jnp.float32),
            jax.ShapeDtypeStruct((B, ENC_DIM), jnp.float32),
            jax.ShapeDtypeStruct((B, N_PAD), jnp.int32),
            jax.ShapeDtypeStruct((B, 16), jnp.float32),
        ],
    )(user_state, W_h, b_h, enc_pad)


def _vs(k, v):
    return plsc.sort_key_val(k, v, descending=True)


def _rev(x):
    return lax.rev(x, (0,))


def _revkv(a):
    return (_rev(a[0]), _rev(a[1]))


def _split(a, b):
    ka, va = a
    kb, vb = b
    mge = ka >= kb
    hi = (jnp.where(mge, ka, kb), jnp.where(mge, va, vb))
    lo = (jnp.where(mge, kb, ka), jnp.where(mge, vb, va))
    return hi, lo


def _clean_keep(C, keep):
    if len(C) == 1:
        return [_vs(*C[0])]
    half = len(C) // 2
    pairs = [_split(C[i], C[i + half]) for i in range(half)]
    hi = [p[0] for p in pairs]
    lo = [p[1] for p in pairs]
    out = _clean_keep(hi, min(keep, half))
    if keep > half:
        out += _clean_keep(lo, keep - half)
    return out


def _merge_top(A, B, keep):
    n = len(A)
    hi, lo = [], []
    for i in range(n):
        h, l = _split(A[i], _revkv(B[n - 1 - i]))
        hi.append(h)
        lo.append(l)
    out = _clean_keep(hi, min(keep, n))
    if keep > n:
        out += _clean_keep(lo, keep - n)
    return out


def _sc_sort_top(kv_list, keep):
    runs = [[_vs(k, v)] for (k, v) in kv_list]
    while len(runs) > 1:
        runs = [_merge_top(runs[i], runs[i + 1], keep)
                for i in range(0, len(runs), 2)]
    return runs[0]


def _sc_kernel_fn(keys_hbm, md_hbm, ids_hbm,
                  asc_hbm, idx_hbm, act_hbm, prb_hbm,
                  kbuf, md_v, ids_v, sk, si,
                  ascs, idxs, acts, prbs):
    c = lax.axis_index("c")
    s = lax.axis_index("s")
    wid = s * 2 + c
    r0 = wid * RPW
    pltpu.sync_copy(md_hbm.at[pl.ds(r0, RPW)], md_v)
    pltpu.sync_copy(ids_hbm, ids_v)
    iota = lax.iota(jnp.int32, L)

    def row_body(r, tile):
        rb = r * N_PAD
        F = []
        for g in range(4):
            f = kbuf[pl.ds(rb + g * 256, L)]
            for j in range(1, 16):
                f = jnp.maximum(f, kbuf[pl.ds(rb + g * 256 + j * L, L)])
            F.append(f)
        s0 = _vs(F[0], F[0])[0]
        s1 = _vs(F[1], F[1])[0]
        s2 = _vs(F[2], F[2])[0]
        s3 = _vs(F[3], F[3])[0]
        X0 = _vs(jnp.maximum(s0, _rev(s1)), F[0])[0]
        X1 = _vs(jnp.minimum(s0, _rev(s1)), F[0])[0]
        Y0 = _vs(jnp.maximum(s2, _rev(s3)), F[0])[0]
        Y1 = _vs(jnp.minimum(s2, _rev(s3)), F[0])[0]
        lo_lo = jnp.minimum(jnp.minimum(X0, _rev(Y1)),
                            jnp.minimum(X1, _rev(Y0)))
        sl = _vs(lo_lo, lo_lo)[0]
        tvec = jnp.full((L,), sl[1], jnp.int32)

        fill_v = jnp.full((L,), N_PAD - 1, jnp.int32)
        for v in range(16):
            si[pl.ds(v * L, L)] = fill_v
        basem1 = jnp.full((L,), -1, jnp.int32)
        UW = 32
        for j0 in range(0, 64, UW):
            kjs = [kbuf[pl.ds(rb + (j0 + u) * L, L)] for u in range(UW)]
            msks = [kj >= tvec for kj in kjs]
            pcs = [plsc.all_reduce_population_count(m) for m in msks]
            css = [plsc.cumsum(m.astype(jnp.int32)) for m in msks]
            offs = [basem1]
            for u in range(UW - 1):
                offs.append(offs[u] + pcs[u])
            for u in range(UW):
                plsc.store_scatter(si, [offs[u] + css[u]],
                                   iota + (j0 + u) * L, mask=msks[u])
            basem1 = offs[UW - 1] + pcs[UW - 1]

        rb_v = jnp.full((L,), rb, jnp.int32)

        def _sort_runs(n):
            def f():
                kv = []
                for i in range(n):
                    vi = si[pl.ds(i * L, L)]
                    ki = plsc.load_gather(kbuf, [rb_v + vi])
                    kv.append((ki, vi))
                top = _sc_sort_top(kv, 4)
                for i in range(4):
                    sk[pl.ds(i * L, L)] = top[i][0]
                    si[pl.ds(i * L, L)] = top[i][1]
            return f

        lax.cond(basem1[0] <= 127, _sort_runs(8), _sort_runs(16))

        eq = None
        for i in range(4):
            ka = sk[pl.ds(i * L, L)]
            nxt = jnp.minimum(iota + jnp.int32(i * L + 1), jnp.int32(63))
            kb = plsc.load_gather(sk, [nxt])
            e = ka == kb
            if i == 3:
                e = e & (iota < jnp.int32(15))
            eq = e if eq is None else (eq | e)
        n_eq = plsc.all_reduce_population_count(eq)

        def _tie_repair():
            for p in (0, 1, 0, 1):
                for h in range(2):
                    ia = (iota + (16 * h)) * 2 + p
                    ib = jnp.minimum(ia + 1, jnp.int32(63))
                    ka = plsc.load_gather(sk, [ia])
                    kb = plsc.load_gather(sk, [ib])
                    va = plsc.load_gather(si, [ia])
                    vb = plsc.load_gather(si, [ib])
                    sw = (ka == kb) & (va > vb)
                    plsc.store_scatter(si, [ia], vb, mask=sw)
                    plsc.store_scatter(si, [ib], va, mask=sw)

        lax.cond(n_eq[0] > 0, _tie_repair, lambda: None)

        row = tile * RT + r
        mdv = md_v[row, pl.ds(0, L)]
        mrow = jnp.full((L,), mdv[0], jnp.float32)
        drow = jnp.full((L,), mdv[1], jnp.float32)
        for i in range(4):
            khi = sk[pl.ds(i * L, L)]
            vhi = si[pl.ds(i * L, L)]
            bts = jnp.where(khi >= 0, khi, khi ^ jnp.int32(0x7FFFFFFF))
            val = lax.bitcast_convert_type(bts, jnp.float32)
            ascs[r, pl.ds(i * L, L)] = val
            idxs[r, pl.ds(i * L, L)] = vhi
            acts[r, pl.ds(i * L, L)] = plsc.load_gather(ids_v, [vhi])
            prbs[r, pl.ds(i * L, L)] = jnp.exp(val - mrow) / drow
        return tile

    def tile_body(t, carry):
        rstart = r0 + t * RT
        pltpu.sync_copy(keys_hbm.at[pl.ds(rstart * N_PAD, RT * N_PAD)], kbuf)
        lax.fori_loop(0, RT, row_body, t)
        pltpu.sync_copy(ascs, asc_hbm.at[pl.ds(rstart, RT)])
        pltpu.sync_copy(idxs, idx_hbm.at[pl.ds(rstart, RT)])
        pltpu.sync_copy(acts, act_hbm.at[pl.ds(rstart, RT)])
        pltpu.sync_copy(prbs, prb_hbm.at[pl.ds(rstart, RT)])
        return carry

    lax.fori_loop(0, NT, tile_body, 0)


@functools.partial(
    pl.kernel,
    out_type=[
        jax.ShapeDtypeStruct((B, 64), jnp.float32),
        jax.ShapeDtypeStruct((B, 64), jnp.int32),
        jax.ShapeDtypeStruct((B, 64), jnp.int32),
        jax.ShapeDtypeStruct((B, 64), jnp.float32),
    ],
    mesh=plsc.VectorSubcoreMesh(core_axis_name="c", subcore_axis_name="s"),
    compiler_params=pltpu.CompilerParams(needs_layout_passes=False),
    scratch_types=[
        pltpu.VMEM((RT * N_PAD,), jnp.int32),
        pltpu.VMEM((RPW, 16), jnp.float32),
        pltpu.VMEM((N_PAD,), jnp.int32),
        pltpu.VMEM((64,), jnp.int32),
        pltpu.VMEM((N_PAD,), jnp.int32),
        pltpu.VMEM((RT, 64), jnp.float32),
        pltpu.VMEM((RT, 64), jnp.int32),
        pltpu.VMEM((RT, 64), jnp.int32),
        pltpu.VMEM((RT, 64), jnp.float32),
    ],
)
def _sc_topk(keys_hbm, md_hbm, ids_hbm,
             asc_hbm, idx_hbm, act_hbm, prb_hbm,
             kbuf, md_v, ids_v, sk, si,
             ascs, idxs, acts, prbs):
    _sc_kernel_fn(keys_hbm, md_hbm, ids_hbm,
                  asc_hbm, idx_hbm, act_hbm, prb_hbm,
                  kbuf, md_v, ids_v, sk, si,
                  ascs, idxs, acts, prbs)


def kernel(user_state, candidate_item_enc, candidate_item_ids, W_h, b_h):
    enc_pad = jnp.zeros((N_PAD, ENC_DIM), jnp.float32).at[:N_CAND].set(
        candidate_item_enc)
    ids_pad = jnp.zeros((N_PAD,), jnp.int32).at[:N_CAND].set(
        candidate_item_ids)
    scores, all_probs, hyper_action, keys, md = _tc_call(
        user_state, W_h, b_h, enc_pad)
    asc, idx, act, prb = _sc_topk(keys.reshape(-1), md, ids_pad)
    return (scores, asc[:, :SLATE], idx[:, :SLATE], act[:, :SLATE],
            all_probs, prb[:, :SLATE], hyper_action)


# speedup vs baseline: 1.0142x; 1.0142x over previous
import functools
import math

import jax
import jax.numpy as jnp
from jax import lax
from jax.experimental import pallas as pl
from jax.experimental.pallas import tpu as pltpu
from jax.experimental.pallas import tpu_sc as plsc

B = 16384
STATE_DIM = 256
ENC_DIM = 128
N_CAND = 1000
N_PAD = 1024
SLATE = 50
BR = 256

_IMIN = -2147483648
L = 16
NW = 32
RPW = B // NW
RT = 8
NT = RPW // RT


def _tc_body(x_ref, w_ref, b_ref, enc_ref,
             scores_ref, probs_ref, ha_ref, keys_ref, md_ref):
    x = x_ref[...]
    ha = jnp.dot(x, w_ref[...], preferred_element_type=jnp.float32)
    ha = ha + b_ref[...][None, :]
    ha_ref[...] = ha
    s = lax.dot_general(
        ha, enc_ref[...], (((1,), (1,)), ((), ())),
        preferred_element_type=jnp.float32,
    ) * (1.0 / math.sqrt(ENC_DIM))
    col = lax.broadcasted_iota(jnp.int32, (BR, N_PAD), 1)
    valid = col < N_CAND
    s_m = jnp.where(valid, s, -jnp.inf)
    m = jnp.max(s_m, axis=1, keepdims=True)
    e = jnp.exp(s_m - m)
    denom = jnp.sum(e, axis=1, keepdims=True)
    probs_ref[...] = (e / denom)[:, :N_CAND]
    scores_ref[...] = s[:, :N_CAND]
    bits = lax.bitcast_convert_type(s, jnp.int32)
    key = jnp.where(bits >= 0, bits, bits ^ jnp.int32(0x7FFFFFFF))
    keys_ref[...] = jnp.where(valid, key, jnp.int32(_IMIN))
    md_ref[...] = jnp.concatenate(
        [m, denom, jnp.zeros((BR, 14), jnp.float32)], axis=1)


def _tc_call(user_state, W_h, b_h, enc_pad):
    return pl.pallas_call(
        _tc_body,
        grid=(B // BR,),
        in_specs=[
            pl.BlockSpec((BR, STATE_DIM), lambda i: (i, 0)),
            pl.BlockSpec((STATE_DIM, ENC_DIM), lambda i: (0, 0)),
            pl.BlockSpec((ENC_DIM,), lambda i: (0,)),
            pl.BlockSpec((N_PAD, ENC_DIM), lambda i: (0, 0)),
        ],
        out_specs=[
            pl.BlockSpec((BR, N_CAND), lambda i: (i, 0)),
            pl.BlockSpec((BR, N_CAND), lambda i: (i, 0)),
            pl.BlockSpec((BR, ENC_DIM), lambda i: (i, 0)),
            pl.BlockSpec((BR, N_PAD), lambda i: (i, 0)),
            pl.BlockSpec((BR, 16), lambda i: (i, 0)),
        ],
        out_shape=[
            jax.ShapeDtypeStruct((B, N_CAND), jnp.float32),
            jax.ShapeDtypeStruct((B, N_CAND), jnp.float32),
            jax.ShapeDtypeStruct((B, ENC_DIM), jnp.float32),
            jax.ShapeDtypeStruct((B, N_PAD), jnp.int32),
            jax.ShapeDtypeStruct((B, 16), jnp.float32),
        ],
    )(user_state, W_h, b_h, enc_pad)


def _vs(k, v):
    return plsc.sort_key_val(k, v, descending=True)


def _rev(x):
    return lax.rev(x, (0,))


def _revkv(a):
    return (_rev(a[0]), _rev(a[1]))


def _split(a, b):
    ka, va = a
    kb, vb = b
    mge = ka >= kb
    hi = (jnp.where(mge, ka, kb), jnp.where(mge, va, vb))
    lo = (jnp.where(mge, kb, ka), jnp.where(mge, vb, va))
    return hi, lo


def _clean_keep(C, keep):
    if len(C) == 1:
        return [_vs(*C[0])]
    half = len(C) // 2
    pairs = [_split(C[i], C[i + half]) for i in range(half)]
    hi = [p[0] for p in pairs]
    lo = [p[1] for p in pairs]
    out = _clean_keep(hi, min(keep, half))
    if keep > half:
        out += _clean_keep(lo, keep - half)
    return out


def _merge_top(A, B, keep):
    n = len(A)
    hi, lo = [], []
    for i in range(n):
        h, l = _split(A[i], _revkv(B[n - 1 - i]))
        hi.append(h)
        lo.append(l)
    out = _clean_keep(hi, min(keep, n))
    if keep > n:
        out += _clean_keep(lo, keep - n)
    return out


def _sc_sort_top(kv_list, keep):
    runs = [[_vs(k, v)] for (k, v) in kv_list]
    while len(runs) > 1:
        runs = [_merge_top(runs[i], runs[i + 1], keep)
                for i in range(0, len(runs), 2)]
    return runs[0]


def _sc_kernel_fn(keys_hbm, md_hbm, ids_hbm,
                  asc_hbm, idx_hbm, act_hbm, prb_hbm,
                  kbuf, md_v, ids_v, sk, si,
                  ascs, idxs, acts, prbs):
    c = lax.axis_index("c")
    s = lax.axis_index("s")
    wid = s * 2 + c
    r0 = wid * RPW
    pltpu.sync_copy(md_hbm.at[pl.ds(r0, RPW)], md_v)
    pltpu.sync_copy(ids_hbm, ids_v)
    iota = lax.iota(jnp.int32, L)

    def row_body(r, tile):
        rb = r * N_PAD
        F = []
        for g in range(4):
            f = kbuf[pl.ds(rb + g * 256, L)]
            for j in range(1, 16):
                f = jnp.maximum(f, kbuf[pl.ds(rb + g * 256 + j * L, L)])
            F.append(f)
        s0 = _vs(F[0], F[0])[0]
        s1 = _vs(F[1], F[1])[0]
        s2 = _vs(F[2], F[2])[0]
        s3 = _vs(F[3], F[3])[0]
        X0 = _vs(jnp.maximum(s0, _rev(s1)), F[0])[0]
        X1 = _vs(jnp.minimum(s0, _rev(s1)), F[0])[0]
        Y0 = _vs(jnp.maximum(s2, _rev(s3)), F[0])[0]
        Y1 = _vs(jnp.minimum(s2, _rev(s3)), F[0])[0]
        lo_lo = jnp.minimum(jnp.minimum(X0, _rev(Y1)),
                            jnp.minimum(X1, _rev(Y0)))
        sl = _vs(lo_lo, lo_lo)[0]
        tvec = jnp.full((L,), sl[1], jnp.int32)

        fill_v = jnp.full((L,), N_PAD - 1, jnp.int32)
        for v in range(16):
            si[pl.ds(v * L, L)] = fill_v
        basem1 = jnp.full((L,), -1, jnp.int32)
        UW = 16
        for j0 in range(0, 64, UW):
            kjs = [kbuf[pl.ds(rb + (j0 + u) * L, L)] for u in range(UW)]
            msks = [kj >= tvec for kj in kjs]
            pcs = [plsc.all_reduce_population_count(m) for m in msks]
            css = [plsc.cumsum(m.astype(jnp.int32)) for m in msks]
            offs = [basem1]
            for u in range(UW - 1):
                offs.append(offs[u] + pcs[u])
            for u in range(UW):
                plsc.store_scatter(si, [offs[u] + css[u]],
                                   iota + (j0 + u) * L, mask=msks[u])
            basem1 = offs[UW - 1] + pcs[UW - 1]

        rb_v = jnp.full((L,), rb, jnp.int32)

        def _sort_runs(n):
            def f():
                kv = []
                for i in range(n):
                    vi = si[pl.ds(i * L, L)]
                    ki = plsc.load_gather(kbuf, [rb_v + vi])
                    kv.append((ki, vi))
                top = _sc_sort_top(kv, 4)
                for i in range(4):
                    sk[pl.ds(i * L, L)] = top[i][0]
                    si[pl.ds(i * L, L)] = top[i][1]
            return f

        lax.cond(basem1[0] <= 127, _sort_runs(8), _sort_runs(16))

        eq = None
        for i in range(4):
            ka = sk[pl.ds(i * L, L)]
            nxt = jnp.minimum(iota + jnp.int32(i * L + 1), jnp.int32(63))
            kb = plsc.load_gather(sk, [nxt])
            e = ka == kb
            if i == 3:
                e = e & (iota < jnp.int32(15))
            eq = e if eq is None else (eq | e)
        n_eq = plsc.all_reduce_population_count(eq)

        def _tie_repair():
            for p in (0, 1, 0, 1):
                for h in range(2):
                    ia = (iota + (16 * h)) * 2 + p
                    ib = jnp.minimum(ia + 1, jnp.int32(63))
                    ka = plsc.load_gather(sk, [ia])
                    kb = plsc.load_gather(sk, [ib])
                    va = plsc.load_gather(si, [ia])
                    vb = plsc.load_gather(si, [ib])
                    sw = (ka == kb) & (va > vb)
                    plsc.store_scatter(si, [ia], vb, mask=sw)
                    plsc.store_scatter(si, [ib], va, mask=sw)

        lax.cond(n_eq[0] > 0, _tie_repair, lambda: None)

        row = tile * RT + r
        mdv = md_v[row, pl.ds(0, L)]
        mrow = jnp.full((L,), mdv[0], jnp.float32)
        drow = jnp.full((L,), mdv[1], jnp.float32)
        for i in range(4):
            khi = sk[pl.ds(i * L, L)]
            vhi = si[pl.ds(i * L, L)]
            bts = jnp.where(khi >= 0, khi, khi ^ jnp.int32(0x7FFFFFFF))
            val = lax.bitcast_convert_type(bts, jnp.float32)
            ascs[r, pl.ds(i * L, L)] = val
            idxs[r, pl.ds(i * L, L)] = vhi
            acts[r, pl.ds(i * L, L)] = plsc.load_gather(ids_v, [vhi])
            prbs[r, pl.ds(i * L, L)] = jnp.exp(val - mrow) / drow
        return tile

    def tile_body(t, carry):
        rstart = r0 + t * RT
        pltpu.sync_copy(keys_hbm.at[pl.ds(rstart * N_PAD, RT * N_PAD)], kbuf)
        lax.fori_loop(0, RT, row_body, t)
        pltpu.sync_copy(ascs, asc_hbm.at[pl.ds(rstart, RT)])
        pltpu.sync_copy(idxs, idx_hbm.at[pl.ds(rstart, RT)])
        pltpu.sync_copy(acts, act_hbm.at[pl.ds(rstart, RT)])
        pltpu.sync_copy(prbs, prb_hbm.at[pl.ds(rstart, RT)])
        return carry

    lax.fori_loop(0, NT, tile_body, 0)


@functools.partial(
    pl.kernel,
    out_type=[
        jax.ShapeDtypeStruct((B, 64), jnp.float32),
        jax.ShapeDtypeStruct((B, 64), jnp.int32),
        jax.ShapeDtypeStruct((B, 64), jnp.int32),
        jax.ShapeDtypeStruct((B, 64), jnp.float32),
    ],
    mesh=plsc.VectorSubcoreMesh(core_axis_name="c", subcore_axis_name="s"),
    compiler_params=pltpu.CompilerParams(needs_layout_passes=False),
    scratch_types=[
        pltpu.VMEM((RT * N_PAD,), jnp.int32),
        pltpu.VMEM((RPW, 16), jnp.float32),
        pltpu.VMEM((N_PAD,), jnp.int32),
        pltpu.VMEM((64,), jnp.int32),
        pltpu.VMEM((N_PAD,), jnp.int32),
        pltpu.VMEM((RT, 64), jnp.float32),
        pltpu.VMEM((RT, 64), jnp.int32),
        pltpu.VMEM((RT, 64), jnp.int32),
        pltpu.VMEM((RT, 64), jnp.float32),
    ],
)
def _sc_topk(keys_hbm, md_hbm, ids_hbm,
             asc_hbm, idx_hbm, act_hbm, prb_hbm,
             kbuf, md_v, ids_v, sk, si,
             ascs, idxs, acts, prbs):
    _sc_kernel_fn(keys_hbm, md_hbm, ids_hbm,
                  asc_hbm, idx_hbm, act_hbm, prb_hbm,
                  kbuf, md_v, ids_v, sk, si,
                  ascs, idxs, acts, prbs)


def kernel(user_state, candidate_item_enc, candidate_item_ids, W_h, b_h):
    enc_pad = jnp.zeros((N_PAD, ENC_DIM), jnp.float32).at[:N_CAND].set(
        candidate_item_enc)
    ids_pad = jnp.zeros((N_PAD,), jnp.int32).at[:N_CAND].set(
        candidate_item_ids)
    scores, all_probs, hyper_action, keys, md = _tc_call(
        user_state, W_h, b_h, enc_pad)
    asc, idx, act, prb = _sc_topk(keys.reshape(-1), md, ids_pad)
    return (scores, asc[:, :SLATE], idx[:, :SLATE], act[:, :SLATE],
            all_probs, prb[:, :SLATE], hyper_action)
